# baseline (speedup 1.0000x reference)
import functools
import jax
import jax.numpy as jnp
from jax import lax
from jax.experimental import pallas as pl
from jax.experimental.pallas import tpu as pltpu
from jax.experimental.pallas import tpu_sc as plsc

_CHUNK = 1024
_DEPTH = 4

SC_ROWS = 4096
_NW = 32
_RPW = SC_ROWS // _NW
_NB = _RPW // 16


def _tc_body(x_hbm, p_hbm, o_ref, x_buf, p_ref, sems, psem):
    n_chunks = (x_hbm.shape[0] - SC_ROWS) // _CHUNK

    p_cp = pltpu.make_async_copy(p_hbm, p_ref, psem)
    p_cp.start()

    def copy(i, slot):
        return pltpu.make_async_copy(
            x_hbm.at[pl.ds(SC_ROWS + i * _CHUNK, _CHUNK), :],
            x_buf.at[slot],
            sems.at[slot],
        )

    for j in range(_DEPTH):
        copy(j, j).start()

    p_cp.wait()
    w1t = p_ref[0:8, :]
    w2t = p_ref[8:16, 0:8]
    thr_col = p_ref[16:24, 0:1]
    for i in range(n_chunks):
        slot = i % _DEPTH
        copy(i, slot).wait()
        ut = lax.dot_general(
            w1t, x_buf[slot],
            (((1,), (1,)), ((), ())),
            preferred_element_type=jnp.float32,
        )
        ht = jnp.tanh(ut - thr_col)
        ot = jnp.tanh(
            lax.dot_general(
                w2t, ht,
                (((1,), (0,)), ((), ())),
                preferred_element_type=jnp.float32,
            )
            - 0.5
        )
        o_ref[:, pl.ds(i * _CHUNK, _CHUNK)] = ot
        if i + _DEPTH < n_chunks:
            copy(i + _DEPTH, slot).start()


def _tc_call(x, packed):
    batch = x.shape[0] - SC_ROWS
    return pl.pallas_call(
        _tc_body,
        in_specs=[
            pl.BlockSpec(memory_space=pl.ANY),
            pl.BlockSpec(memory_space=pl.ANY),
        ],
        out_specs=pl.BlockSpec(memory_space=pltpu.VMEM),
        out_shape=jax.ShapeDtypeStruct((8, batch), jnp.float32),
        scratch_shapes=[
            pltpu.VMEM((_DEPTH, _CHUNK, x.shape[1]), jnp.float32),
            pltpu.VMEM((48, 512), jnp.float32),
            pltpu.SemaphoreType.DMA((_DEPTH,)),
            pltpu.SemaphoreType.DMA,
        ],
    )(x, packed)


def _sc_body(x_hbm, p_hbm, o_hbm, blockbuf, pbuf, tmpbuf):
    wid = lax.axis_index("s") * 2 + lax.axis_index("c")
    row0 = wid * _RPW
    pltpu.sync_copy(p_hbm, pbuf)

    def _tanh16(z):
        return 1.0 - 2.0 / (jnp.exp(2.0 * z) + 1.0)

    iota = lax.iota(jnp.int32, 16)
    dn = lax.GatherDimensionNumbers(
        offset_dims=(), collapsed_slice_dims=(0,), start_index_map=(0,)
    )

    def lanesum(v):
        for k in (1, 2, 4, 8):
            v = v + lax.gather(
                v, (iota ^ k)[:, None], dn, (1,),
                mode=lax.GatherScatterMode.PROMISE_IN_BOUNDS,
            )
        return v

    thrv = pbuf[24, 0:16]
    w2v = [pbuf[8 + o, 0:16] for o in range(4)]
    maskv = pbuf[25, 0:16]
    onehot = [pbuf[32 + r, 0:16] for r in range(16)]
    consts = [lanesum(_tanh16(-thrv) * maskv * w2v[o]) for o in range(4)]
    thr_b = [lanesum(thrv * onehot[j]) for j in range(3)]
    w2_b = [[lanesum(w2v[o] * onehot[j]) for j in range(3)] for o in range(4)]

    def block_step(b, carry):
        pltpu.sync_copy(x_hbm.at[pl.ds(row0 + b * 16, 16), :], blockbuf)
        uvec = [None, None, None]
        for r in range(16):
            acc = [None, None, None]
            for c in range(32):
                xv = blockbuf[r, 16 * c:16 * c + 16]
                for j in range(3):
                    t = xv * pbuf[j, 16 * c:16 * c + 16]
                    acc[j] = t if acc[j] is None else acc[j] + t
            for j in range(3):
                u = lanesum(acc[j]) * onehot[r]
                uvec[j] = u if uvec[j] is None else uvec[j] + u
        hs = [_tanh16(uvec[j] - thr_b[j]) for j in range(3)]
        for o in range(4):
            z = (
                hs[0] * w2_b[o][0]
                + hs[1] * w2_b[o][1]
                + hs[2] * w2_b[o][2]
                + (consts[o] - 0.5)
            )
            tmpbuf[o, 0:16] = _tanh16(z)
        pltpu.sync_copy(tmpbuf, o_hbm.at[wid, b])
        return carry

    lax.fori_loop(0, _NB, block_step, 0)


def _sc_call(x, packed):
    mesh = plsc.VectorSubcoreMesh(core_axis_name="c", subcore_axis_name="s")
    f = functools.partial(
        pl.kernel,
        mesh=mesh,
        out_type=jax.ShapeDtypeStruct((_NW, _NB, 8, 16), jnp.float32),
        scratch_types=[
            pltpu.VMEM((16, 512), jnp.float32),
            pltpu.VMEM((48, 512), jnp.float32),
            pltpu.VMEM((8, 16), jnp.float32),
        ],
    )(_sc_body)
    return f(x, packed)


def kernel(x, W1, thr_h, W2):
    hidden = W1.shape[1]
    out_size = W2.shape[1]

    packed = jnp.zeros((48, 512), jnp.float32)
    packed = packed.at[0:hidden, :].set(W1.T)
    packed = packed.at[8:8 + out_size, :hidden].set(W2.T)
    packed = packed.at[16:16 + hidden, 0].set(thr_h)
    packed = packed.at[24, :hidden].set(thr_h)
    packed = packed.at[25, 3:hidden].set(1.0)
    packed = packed.at[32:48, :16].set(jnp.eye(16, dtype=jnp.float32))

    sc4 = _sc_call(x, packed)
    tc_t = _tc_call(x, packed)
    sc_part = sc4.transpose(0, 1, 3, 2).reshape(SC_ROWS, 8)[:, :out_size]
    return jnp.concatenate([sc_part, tc_t[:out_size].T], axis=0)


# speedup vs baseline: 3.5794x; 3.5794x over previous
import jax
import jax.numpy as jnp
from jax import lax
from jax.experimental import pallas as pl
from jax.experimental.pallas import tpu as pltpu

_CHUNK = 1024
_DEPTH = 4


def _body(x_hbm, p_hbm, o_ref, x_buf, p_ref, sems, psem):
    n_chunks = x_hbm.shape[0] // _CHUNK

    p_cp = pltpu.make_async_copy(p_hbm, p_ref, psem)
    p_cp.start()

    def copy(i, slot):
        return pltpu.make_async_copy(
            x_hbm.at[pl.ds(i * _CHUNK, _CHUNK), :],
            x_buf.at[slot],
            sems.at[slot],
        )

    for j in range(_DEPTH):
        copy(j, j).start()

    p_cp.wait()
    w1t = p_ref[0:8, :]
    w2t = p_ref[8:16, 0:8]
    thr_col = p_ref[16:24, 0:1]
    for i in range(n_chunks):
        slot = i % _DEPTH
        copy(i, slot).wait()
        ut = lax.dot_general(
            w1t, x_buf[slot],
            (((1,), (1,)), ((), ())),
            preferred_element_type=jnp.float32,
        )
        ht = jnp.tanh(ut - thr_col)
        ot = jnp.tanh(
            lax.dot_general(
                w2t, ht,
                (((1,), (0,)), ((), ())),
                preferred_element_type=jnp.float32,
            )
            - 0.5
        )
        o_ref[:, pl.ds(i * _CHUNK, _CHUNK)] = ot
        if i + _DEPTH < n_chunks:
            copy(i + _DEPTH, slot).start()


def kernel(x, W1, thr_h, W2):
    batch, in_size = x.shape
    hidden = W1.shape[1]
    out_size = W2.shape[1]

    packed = jnp.zeros((24, 512), jnp.float32)
    packed = packed.at[0:hidden, :].set(W1.T)
    packed = packed.at[8:8 + out_size, :hidden].set(W2.T)
    packed = packed.at[16:16 + hidden, 0].set(thr_h)

    res_t = pl.pallas_call(
        _body,
        in_specs=[
            pl.BlockSpec(memory_space=pl.ANY),
            pl.BlockSpec(memory_space=pl.ANY),
        ],
        out_specs=pl.BlockSpec(memory_space=pltpu.VMEM),
        out_shape=jax.ShapeDtypeStruct((8, batch), jnp.float32),
        scratch_shapes=[
            pltpu.VMEM((_DEPTH, _CHUNK, in_size), jnp.float32),
            pltpu.VMEM((24, 512), jnp.float32),
            pltpu.SemaphoreType.DMA((_DEPTH,)),
            pltpu.SemaphoreType.DMA,
        ],
    )(x, packed)
    return res_t[:out_size].T
